# Initial kernel scaffold; baseline (speedup 1.0000x reference)
import jax
import jax.numpy as jnp
from jax.experimental import pallas as pl


def kernel(x, edge_index, W_src, b_src, W_fc, b_fc):
    raise NotImplementedError("write your pallas kernel here")


# speedup vs baseline: 6.1232x; 6.1232x over previous
import functools

import jax
import jax.numpy as jnp
from jax import lax
from jax.experimental import pallas as pl
from jax.experimental.pallas import tpu as pltpu
from jax.experimental.pallas import tpu_sc as plsc

NUM_WORKERS = 32
CHUNK = 128
LANES = 16


def _sc_segment_sum(x, packed_w, n_pad, d_in):
    n_chunks = packed_w.shape[1]
    rows_per_tile = n_pad // 16
    deg_rows = n_pad // CHUNK + 1

    mesh = plsc.VectorSubcoreMesh(core_axis_name="c", subcore_axis_name="s")

    @functools.partial(
        pl.kernel,
        out_type=[
            jax.ShapeDtypeStruct((2, n_pad, d_in), jnp.float32),
            jax.ShapeDtypeStruct((2, deg_rows, CHUNK), jnp.float32),
        ],
        mesh=mesh,
        compiler_params=pltpu.CompilerParams(needs_layout_passes=False),
        scratch_types=[
            pltpu.VMEM((n_chunks, CHUNK), jnp.int32),
            pltpu.VMEM((CHUNK,), jnp.int32),
            pltpu.VMEM((CHUNK,), jnp.int32),
            pltpu.VMEM((CHUNK, d_in), jnp.float32),
            pltpu.VMEM((deg_rows, CHUNK), jnp.float32),
            pltpu.VMEM((deg_rows,), jnp.int32),
            pltpu.VMEM_SHARED((n_pad, d_in), jnp.float32),
            pltpu.VMEM_SHARED((deg_rows, CHUNK), jnp.float32),
            pltpu.SemaphoreType.DMA,
        ],
    )
    def sc_kernel(x_hbm, packed_hbm, zrow_hbm, zdeg_hbm,
                  agg_out, deg_out,
                  packed_v, src_v, dst_v, rows_v, deg_v, iota_v, agg_sh,
                  deg_sh, sem):
        c = lax.axis_index("c")
        s = lax.axis_index("s")
        wid = s * 2 + c

        pltpu.sync_copy(packed_hbm.at[wid], packed_v)
        pltpu.sync_copy(zdeg_hbm, deg_v)
        iota16 = lax.iota(jnp.int32, LANES)
        for g in range(deg_rows // LANES + 1):
            if g * LANES + LANES <= deg_rows:
                iota_v[pl.ds(g * LANES, LANES)] = iota16 + g * LANES

        cnt0, _ = plsc.scan_count(jnp.zeros((LANES,), jnp.int32))
        delta = LANES - jnp.max(cnt0)

        r0 = s * rows_per_tile
        pltpu.sync_copy(zrow_hbm.at[pl.ds(r0, rows_per_tile)],
                        agg_sh.at[pl.ds(r0, rows_per_tile)])

        @pl.when(s == 0)
        def _():
            pltpu.sync_copy(zdeg_hbm, deg_sh)

        plsc.subcore_barrier()

        def body(j, carry):
            def unpack(g, carry2):
                packed = packed_v[j, pl.ds(g * LANES, LANES)]
                src16 = packed & 0xFFFF
                dst16 = lax.shift_right_logical(packed, 16)
                src_v[pl.ds(g * LANES, LANES)] = src16
                dst_v[pl.ds(g * LANES, LANES)] = dst16
                cnt, last = plsc.scan_count(dst16)
                plsc.addupdate_scatter(
                    deg_v,
                    [lax.shift_right_logical(dst16, 7), dst16 & 0x7F],
                    (cnt + delta).astype(jnp.float32),
                    mask=last,
                )
                return carry2

            lax.fori_loop(0, CHUNK // LANES, unpack, 0)
            pltpu.async_copy(x_hbm.at[src_v], rows_v, sem).wait()
            pltpu.sync_copy(rows_v, agg_sh.at[dst_v], add=True)
            return carry

        lax.fori_loop(0, n_chunks, body, 0)
        pltpu.sync_copy(deg_v, deg_sh.at[iota_v], add=True)
        plsc.subcore_barrier()

        pltpu.sync_copy(agg_sh.at[pl.ds(r0, rows_per_tile)],
                        agg_out.at[c, pl.ds(r0, rows_per_tile)])

        @pl.when(s == 0)
        def _():
            pltpu.sync_copy(deg_sh, deg_out.at[c])

    zrow = jnp.zeros((n_pad, d_in), jnp.float32)
    zdeg = jnp.zeros((deg_rows, CHUNK), jnp.float32)
    return sc_kernel(x, packed_w, zrow, zdeg)


def _tc_finish(x, a0, a1, d0, d1, W_src, b_src, W_fc, b_fc, blk):
    n, d_in = x.shape
    d_out = W_fc.shape[1]

    def tc_kernel(x_ref, a0_ref, a1_ref, d0_ref, d1_ref,
                  ws_ref, bs_ref, wf_ref, bf_ref, o_ref):
        deg = d0_ref[...] + d1_ref[...]
        degc = jnp.maximum(deg, 1.0)
        agg = a0_ref[...] + a1_ref[...]
        msum = jnp.dot(agg, ws_ref[...], preferred_element_type=jnp.float32)
        msum = msum + deg * bs_ref[...]
        neigh = msum / degc
        wf = wf_ref[...]
        h = jnp.dot(x_ref[...], wf[:d_in], preferred_element_type=jnp.float32)
        h = h + jnp.dot(neigh, wf[d_in:], preferred_element_type=jnp.float32)
        o_ref[...] = h + bf_ref[...]

    return pl.pallas_call(
        tc_kernel,
        grid=(n // blk,),
        in_specs=[
            pl.BlockSpec((blk, d_in), lambda i: (i, 0)),
            pl.BlockSpec((blk, d_in), lambda i: (i, 0)),
            pl.BlockSpec((blk, d_in), lambda i: (i, 0)),
            pl.BlockSpec((blk, 1), lambda i: (i, 0)),
            pl.BlockSpec((blk, 1), lambda i: (i, 0)),
            pl.BlockSpec((d_in, d_in), lambda i: (0, 0)),
            pl.BlockSpec((1, d_in), lambda i: (0, 0)),
            pl.BlockSpec((2 * d_in, d_out), lambda i: (0, 0)),
            pl.BlockSpec((1, d_out), lambda i: (0, 0)),
        ],
        out_specs=pl.BlockSpec((blk, d_out), lambda i: (i, 0)),
        out_shape=jax.ShapeDtypeStruct((n, d_out), jnp.float32),
    )(x, a0, a1, d0, d1, W_src, b_src.reshape(1, d_in), W_fc,
      b_fc.reshape(1, d_out))


def kernel(x, edge_index, W_src, b_src, W_fc, b_fc):
    n, d_in = x.shape
    e = edge_index.shape[1]
    assert e % NUM_WORKERS == 0 and n < (1 << 16)
    ew = e // NUM_WORKERS
    n_chunks = -(-ew // CHUNK)
    ewp = n_chunks * CHUNK
    n_pad = -(-(n + 1) // 128) * 128

    src = edge_index[0].astype(jnp.int32)
    dst = edge_index[1].astype(jnp.int32)
    packed = src | (dst << 16)
    pad_word = jnp.int32(n << 16)
    packed_w = jnp.concatenate(
        [packed.reshape(NUM_WORKERS, ew),
         jnp.full((NUM_WORKERS, ewp - ew), pad_word, jnp.int32)], axis=1
    ).reshape(NUM_WORKERS, n_chunks, CHUNK)

    agg, degw = _sc_segment_sum(x, packed_w, n_pad, d_in)
    deg0 = degw[0].reshape(-1, 1)[:n]
    deg1 = degw[1].reshape(-1, 1)[:n]

    blk = 1000 if n % 1000 == 0 else 8
    return _tc_finish(
        x, agg[0, :n], agg[1, :n], deg0, deg1,
        W_src, b_src, W_fc, b_fc, blk,
    )
